# baseline (speedup 1.0000x reference)
import functools

import jax
import jax.numpy as jnp
from jax.experimental import pallas as pl
from jax.experimental.pallas import tpu as pltpu

_ALPHA = 0.2
_STREAMS = 5
_BI = 80


def _fused_kernel(*refs, bi, streams):
    x_ref, w_ref, a_ref = refs[:3]
    adj_refs = refs[3:3 + streams]
    out_ref = refs[3 + streams]
    hb_ref, p_ref, r_ref, qt_ref, tt_ref = refs[4 + streams:]
    i = pl.program_id(0)
    d = w_ref.shape[1]

    @pl.when(i == 0)
    def _prologue():
        h = jnp.dot(x_ref[...], w_ref[...], preferred_element_type=jnp.float32)
        hb_ref[:, :] = jnp.zeros_like(hb_ref)
        hb_ref[:, :d] = h.astype(jnp.bfloat16)
        hb_ref[:, d:d + 1] = jnp.ones((h.shape[0], 1), jnp.bfloat16)
        f = jnp.sum(h * a_ref[0:1, :d], axis=1, keepdims=True)
        p_ref[...] = jnp.exp(-f)
        r_ref[...] = jnp.exp(-_ALPHA * f)
        gt = jax.lax.dot_general(a_ref[:, d:], h, (((1,), (1,)), ((), ())),
                                 preferred_element_type=jnp.float32)
        qt_ref[...] = jnp.exp(-gt)
        tt_ref[...] = jnp.exp(-_ALPHA * gt)

    @pl.when(i > 0)
    def _main():
        k = i - 1
        for s in range(streams):
            base = (k * streams + s) * bi
            pb = p_ref[pl.ds(base, bi)]
            rb = r_ref[pl.ds(base, bi)]
            e = (adj_refs[s][...] * jnp.minimum(pb * qt_ref[...],
                                                rb * tt_ref[...])
                 ).astype(jnp.bfloat16)
            acc = jnp.dot(e, hb_ref[...], preferred_element_type=jnp.float32)
            hp = acc[:, :d] / acc[:, d:d + 1]
            out_ref[s * bi:(s + 1) * bi] = jnp.where(hp > 0, hp,
                                                     jnp.exp(hp) - 1.0)


def kernel(x, adj, w, a):
    n, d_in = x.shape
    d = w.shape[1]

    if n % (_STREAMS * _BI) == 0:
        streams, bi = _STREAMS, _BI
    else:
        streams, bi = 1, n
    ni = n // (streams * bi)

    def _adj_spec(s):
        return pl.BlockSpec(
            (bi, n), lambda i, s=s: (streams * jnp.maximum(i - 1, 0) + s, 0))

    out = pl.pallas_call(
        functools.partial(_fused_kernel, bi=bi, streams=streams),
        grid=(ni + 1,),
        in_specs=[
            pl.BlockSpec((n, d_in), lambda i: (0, 0)),
            pl.BlockSpec((d_in, d), lambda i: (0, 0)),
            pl.BlockSpec((1, 2 * d), lambda i: (0, 0)),
        ] + [_adj_spec(s) for s in range(streams)],
        out_specs=pl.BlockSpec((streams * bi, d),
                               lambda i: (jnp.maximum(i - 1, 0), 0)),
        out_shape=jax.ShapeDtypeStruct((n, d), jnp.float32),
        scratch_shapes=[
            pltpu.VMEM((n, 2 * d), jnp.bfloat16),
            pltpu.VMEM((n, 1), jnp.float32),
            pltpu.VMEM((n, 1), jnp.float32),
            pltpu.VMEM((1, n), jnp.float32),
            pltpu.VMEM((1, n), jnp.float32),
        ],
    )(x, w, a, *([adj] * streams))
    return out


# speedup vs baseline: 1.0288x; 1.0288x over previous
import functools

import jax
import jax.numpy as jnp
from jax.experimental import pallas as pl
from jax.experimental.pallas import tpu as pltpu

_ALPHA = 0.2
_STREAMS = 2
_BI = 200


def _fused_kernel(*refs, bi, streams):
    x_ref, w_ref, a_ref = refs[:3]
    adj_refs = refs[3:3 + streams]
    out_ref = refs[3 + streams]
    hb_ref, p_ref, r_ref, qt_ref, tt_ref = refs[4 + streams:]
    i = pl.program_id(0)
    d = w_ref.shape[1]

    @pl.when(i == 0)
    def _prologue():
        h = jnp.dot(x_ref[...], w_ref[...], preferred_element_type=jnp.float32)
        hb_ref[:, :] = jnp.zeros_like(hb_ref)
        hb_ref[:, :d] = h.astype(jnp.bfloat16)
        hb_ref[:, d:d + 1] = jnp.ones((h.shape[0], 1), jnp.bfloat16)
        f = jnp.sum(h * a_ref[0:1, :d], axis=1, keepdims=True)
        p_ref[...] = jnp.exp(-f)
        r_ref[...] = jnp.exp(-_ALPHA * f)
        gt = jax.lax.dot_general(a_ref[:, d:], h, (((1,), (1,)), ((), ())),
                                 preferred_element_type=jnp.float32)
        qt_ref[...] = jnp.exp(-gt)
        tt_ref[...] = jnp.exp(-_ALPHA * gt)

    @pl.when(i > 0)
    def _main():
        k = i - 1
        for s in range(streams):
            base = (k * streams + s) * bi
            pb = p_ref[pl.ds(base, bi)]
            rb = r_ref[pl.ds(base, bi)]
            e = (adj_refs[s][...] * jnp.minimum(pb * qt_ref[...],
                                                rb * tt_ref[...])
                 ).astype(jnp.bfloat16)
            acc = jnp.dot(e, hb_ref[...], preferred_element_type=jnp.float32)
            hp = acc[:, :d] / acc[:, d:d + 1]
            out_ref[s * bi:(s + 1) * bi] = jnp.where(hp > 0, hp,
                                                     jnp.exp(hp) - 1.0)


def kernel(x, adj, w, a):
    n, d_in = x.shape
    d = w.shape[1]

    if n % (_STREAMS * _BI) == 0:
        streams, bi = _STREAMS, _BI
    else:
        streams, bi = 1, n
    ni = n // (streams * bi)

    def _adj_spec(s):
        return pl.BlockSpec(
            (bi, n), lambda i, s=s: (streams * jnp.maximum(i - 1, 0) + s, 0))

    out = pl.pallas_call(
        functools.partial(_fused_kernel, bi=bi, streams=streams),
        grid=(ni + 1,),
        in_specs=[
            pl.BlockSpec((n, d_in), lambda i: (0, 0)),
            pl.BlockSpec((d_in, d), lambda i: (0, 0)),
            pl.BlockSpec((1, 2 * d), lambda i: (0, 0)),
        ] + [_adj_spec(s) for s in range(streams)],
        out_specs=pl.BlockSpec((streams * bi, d),
                               lambda i: (jnp.maximum(i - 1, 0), 0)),
        out_shape=jax.ShapeDtypeStruct((n, d), jnp.float32),
        scratch_shapes=[
            pltpu.VMEM((n, 2 * d), jnp.bfloat16),
            pltpu.VMEM((n, 1), jnp.float32),
            pltpu.VMEM((n, 1), jnp.float32),
            pltpu.VMEM((1, n), jnp.float32),
            pltpu.VMEM((1, n), jnp.float32),
        ],
    )(x, w, a, *([adj] * streams))
    return out


# speedup vs baseline: 1.0307x; 1.0018x over previous
import functools

import jax
import jax.numpy as jnp
from jax.experimental import pallas as pl
from jax.experimental.pallas import tpu as pltpu

_ALPHA = 0.2
_STREAMS = 2
_BI = 200


def _fused_kernel(*refs, bi, streams, half):
    x_ref, w_ref, a_ref = refs[:3]
    adj_refs = refs[3:3 + streams]
    out_ref = refs[3 + streams]
    hb_ref, p_ref, r_ref, qt_ref, tt_ref = refs[4 + streams:]
    i = pl.program_id(0)
    d = w_ref.shape[1]

    @pl.when(i == 0)
    def _prologue():
        h = jnp.dot(x_ref[...], w_ref[...], preferred_element_type=jnp.float32)
        hb_ref[:, :] = jnp.zeros_like(hb_ref)
        hb_ref[:, :d] = h.astype(jnp.bfloat16)
        hb_ref[:, d:d + 1] = jnp.ones((h.shape[0], 1), jnp.bfloat16)
        f = jnp.sum(h * a_ref[0:1, :d], axis=1, keepdims=True)
        p_ref[...] = jnp.exp(-f)
        r_ref[...] = jnp.exp(-_ALPHA * f)
        gt = jax.lax.dot_general(a_ref[:, d:], h, (((1,), (1,)), ((), ())),
                                 preferred_element_type=jnp.float32)
        qt_ref[...] = jnp.exp(-gt)
        tt_ref[...] = jnp.exp(-_ALPHA * gt)

    @pl.when(i > 0)
    def _main():
        k = i - 1
        for s in range(streams):
            base = (s * half + k) * bi
            pb = p_ref[pl.ds(base, bi)]
            rb = r_ref[pl.ds(base, bi)]
            e = (adj_refs[s][...] * jnp.minimum(pb * qt_ref[...],
                                                rb * tt_ref[...])
                 ).astype(jnp.bfloat16)
            acc = jnp.dot(e, hb_ref[...], preferred_element_type=jnp.float32)
            hp = acc[:, :d] / acc[:, d:d + 1]
            out_ref[s, :, :] = jnp.where(hp > 0, hp, jnp.exp(hp) - 1.0)


def kernel(x, adj, w, a):
    n, d_in = x.shape
    d = w.shape[1]

    if n % (_STREAMS * _BI) == 0:
        streams, bi = _STREAMS, _BI
    else:
        streams, bi = 1, n
    ni = n // (streams * bi)

    def _adj_spec(s):
        return pl.BlockSpec(
            (bi, n), lambda i, s=s: (jnp.maximum(i - 1, 0) + s * ni, 0))

    out = pl.pallas_call(
        functools.partial(_fused_kernel, bi=bi, streams=streams, half=ni),
        grid=(ni + 1,),
        in_specs=[
            pl.BlockSpec((n, d_in), lambda i: (0, 0)),
            pl.BlockSpec((d_in, d), lambda i: (0, 0)),
            pl.BlockSpec((1, 2 * d), lambda i: (0, 0)),
        ] + [_adj_spec(s) for s in range(streams)],
        out_specs=pl.BlockSpec((streams, bi, d),
                               lambda i: (0, jnp.maximum(i - 1, 0), 0)),
        out_shape=jax.ShapeDtypeStruct((streams, n // streams, d),
                                       jnp.float32),
        scratch_shapes=[
            pltpu.VMEM((n, 2 * d), jnp.bfloat16),
            pltpu.VMEM((n, 1), jnp.float32),
            pltpu.VMEM((n, 1), jnp.float32),
            pltpu.VMEM((1, n), jnp.float32),
            pltpu.VMEM((1, n), jnp.float32),
        ],
    )(x, w, a, *([adj] * streams))
    return out.reshape(n, d)
